# baseline (speedup 1.0000x reference)
import functools

import jax
import jax.numpy as jnp
from jax import lax
from jax.experimental import pallas as pl
from jax.experimental.pallas import tpu as pltpu
from jax.experimental.pallas import tpu_sc as plsc

_LAMDA = 1e-05
_SLOPE = 0.01

_NC = 2
_NS = 16
_NW = _NC * _NS
_CHUNK = 64


def _leaky(x):
    return jnp.where(x >= 0, x, _SLOPE * x)


_NBUF = 8
_CB = 120


def _sc_gather_all(table, idx_p, idx_b, r_a, r_b):
    p = len(idx_p)
    d = table.shape[1]
    rw_a = r_a // _NW
    nch_a = rw_a // _CHUNK
    rw_b = r_b // _NW
    nch_b = rw_b // _CB
    assert nch_a % _NBUF == 0 and nch_b % _NBUF == 0

    mesh = plsc.VectorSubcoreMesh(
        core_axis_name="c", subcore_axis_name="s",
        num_cores=_NC, num_subcores=_NS,
    )

    @functools.partial(
        pl.kernel,
        out_type=(
            [jax.ShapeDtypeStruct((r_a, d), table.dtype)] * p
            + [jax.ShapeDtypeStruct((r_b, d), table.dtype)]
        ),
        mesh=mesh,
        scratch_types=(
            [pltpu.VMEM((p * rw_a,), jnp.int32),
             pltpu.VMEM((rw_b,), jnp.int32)]
            + [pltpu.VMEM((_CHUNK, d), table.dtype)] * _NBUF
            + [pltpu.SemaphoreType.DMA] * (2 * _NBUF)
        ),
    )
    def gather_k(table_hbm, *rest):
        idxp_hbm = rest[:p]
        idxb_hbm = rest[p]
        outp_hbm = rest[p + 1:2 * p + 1]
        outb_hbm = rest[2 * p + 1]
        sc = rest[2 * p + 2:]
        idxa_v, idxb_v = sc[0], sc[1]
        bufs = sc[2:2 + _NBUF]
        semg = sc[2 + _NBUF:2 + 2 * _NBUF]
        semw = sc[2 + 2 * _NBUF:]
        wid = lax.axis_index("s") * _NC + lax.axis_index("c")
        base_a = wid * rw_a
        base_b = wid * rw_b

        for j in range(p):
            pltpu.sync_copy(idxp_hbm[j].at[pl.ds(base_a, rw_a)],
                            idxa_v.at[pl.ds(j * rw_a, rw_a)])
        pltpu.sync_copy(idxb_hbm.at[pl.ds(base_b, rw_b)], idxb_v)

        def pipe(nch, gather_mk, write_mk):
            ng = nch // _NBUF
            for b in range(_NBUF):
                gather_mk(b, b).start()

            def body(g, carry):
                for b in range(_NBUF):
                    ch = g * _NBUF + b
                    gather_mk(ch, b).wait()
                    write_mk(ch, b).start()

                @pl.when(g < ng - 1)
                def _():
                    for b in range(_NBUF):
                        ch = g * _NBUF + b
                        write_mk(ch, b).wait()
                        gather_mk(ch + _NBUF, b).start()

                return carry

            lax.fori_loop(0, ng, body, 0)
            for b in range(_NBUF):
                write_mk((ng - 1) * _NBUF + b, b).wait()

        for j in range(p):
            def ga(ch, b, j=j):
                return pltpu.make_async_copy(
                    table_hbm.at[idxa_v.at[
                        pl.ds(j * rw_a + ch * _CHUNK, _CHUNK)]],
                    bufs[b], semg[b])

            def wa(ch, b, j=j):
                return pltpu.make_async_copy(
                    bufs[b],
                    outp_hbm[j].at[pl.ds(base_a + ch * _CHUNK, _CHUNK)],
                    semw[b])

            pipe(nch_a, ga, wa)

        def gb(ch, b):
            return pltpu.make_async_copy(
                table_hbm.at[idxb_v.at[pl.ds(ch * _CB, _CB)]],
                bufs[b].at[pl.ds(0, _CB)], semg[b])

        def wb(ch, b):
            return pltpu.make_async_copy(
                bufs[b].at[pl.ds(0, _CB)],
                outb_hbm.at[pl.ds(base_b + ch * _CB, _CB)], semw[b])

        pipe(nch_b, gb, wb)

    return gather_k(table, *idx_p, idx_b)


def _fused_body(bb, k, p, t, f, nb, b_total, *refs):
    (raw_refs, (hl_ref, feat_ref, m_ref, lp_ref,
                wt_ref, bt_ref,
                wg_ref, bg_ref, wb_ref, bbias_ref, wav_ref, bav_ref,
                wac_ref, bac_ref, wgate_ref, bgate_ref, wpa_ref,
                ww_ref, bw_ref, out_ref, tout_ref, l_ref)) = (
        refs[:p], refs[p:])
    i = pl.program_id(0)
    bbk = bb * k
    f32 = jnp.float32

    wt = wt_ref[...]
    bt = bt_ref[...]
    wtpa = jnp.dot(wt, wpa_ref[...], preferred_element_type=f32)
    segs = [r[...] for r in raw_refs]
    scores = jnp.concatenate(
        [jnp.dot(s, wtpa, preferred_element_type=f32) for s in segs], axis=1
    )
    smax = jnp.max(scores, axis=1, keepdims=True)
    e = jnp.exp(scores - smax)
    pa = e / jnp.sum(e, axis=1, keepdims=True)
    m = m_ref[...]
    w = pa * m
    z = segs[0] * w[:, 0:1]
    for j in range(1, p):
        z = z + segs[j] * w[:, j:j + 1]
    wsum = jnp.sum(w, axis=1, keepdims=True)
    t_p = (jnp.dot(z, wt, preferred_element_type=f32) + bt * wsum) / (
        jnp.sum(m, axis=1, keepdims=True) + 1e-10)

    gamma = _leaky(jnp.dot(t_p, wg_ref[...], preferred_element_type=f32)
                   + bg_ref[...])
    beta = _leaky(jnp.dot(t_p, wb_ref[...], preferred_element_type=f32)
                  + bbias_ref[...])
    hl = hl_ref[...]
    p_x = (gamma + 1.0) * hl + beta

    gsel = (lax.broadcasted_iota(jnp.int32, (bb, bbk), 1) // k
            == lax.broadcasted_iota(jnp.int32, (bb, bbk), 0)).astype(f32)
    rsel = (lax.broadcasted_iota(jnp.int32, (bbk, bb), 0) // k
            == lax.broadcasted_iota(jnp.int32, (bbk, bb), 1)).astype(f32)
    onek = (lax.broadcasted_iota(jnp.int32, (bbk, k), 0) % k
            == lax.broadcasted_iota(jnp.int32, (bbk, k), 1)).astype(f32)
    tsel = (lax.broadcasted_iota(jnp.int32, (k, bbk), 1) % k
            == lax.broadcasted_iota(jnp.int32, (k, bbk), 0)).astype(f32)

    gsum = jnp.dot(gsel, gamma * gamma, preferred_element_type=f32)
    bsum = jnp.dot(gsel, beta * beta, preferred_element_type=f32)
    partial = jnp.sum(jnp.sqrt(gsum)) + jnp.sum(jnp.sqrt(bsum))

    feat = feat_ref[...]
    bav = bav_ref[...]
    q = jnp.dot(feat, wav_ref[...], preferred_element_type=f32) + bav
    katt = jnp.dot(p_x, wav_ref[...], preferred_element_type=f32) + bav
    q_rep = jnp.dot(rsel, q, preferred_element_type=f32)
    s_col = jnp.sum(q_rep * katt, axis=1, keepdims=True)
    s_mat = jnp.dot(gsel, s_col * onek, preferred_element_type=f32)
    amax = jnp.max(s_mat, axis=1, keepdims=True)
    ae = jnp.exp(s_mat - amax)
    aw = ae / jnp.sum(ae, axis=1, keepdims=True)

    alpha = jnp.exp(-_LAMDA * lp_ref[...])
    waw = jnp.dot(aw, tsel, preferred_element_type=f32) * gsel
    wal = jnp.dot(alpha, tsel, preferred_element_type=f32) * gsel
    attn_in = jnp.dot(waw, p_x, preferred_element_type=f32)
    a_x = jnp.dot(wal, p_x, preferred_element_type=f32)

    attn_output = jnp.dot(attn_in, wac_ref[...],
                          preferred_element_type=f32) + bac_ref[...]
    g1 = wgate_ref[0:f, :]
    g2 = wgate_ref[f:2 * f, :]
    gate = jax.nn.sigmoid(
        jnp.dot(a_x, g1, preferred_element_type=f32)
        + jnp.dot(attn_output, g2, preferred_element_type=f32)
        + bgate_ref[...]
    )
    fused = gate * a_x + (1.0 - gate) * attn_output
    update = feat + fused
    out_ref[...] = _leaky(jnp.dot(update, ww_ref[...],
                                  preferred_element_type=f32) + bw_ref[...])
    tout_ref[...] = jnp.dot(feat, wt, preferred_element_type=f32) + bt

    prev = jnp.where(i == 0, jnp.zeros((1, 1), f32), l_ref[...])
    tot = prev + partial
    l_ref[...] = jnp.where(i == nb - 1, tot / b_total, tot)


def _fused(out_p, out_b, masks2, lp, wt, bt, wg, bg, wb, bbias,
           wav, bav, wac, bac, wgate, bgate, wpa, ww, bw):
    b, k = lp.shape
    p = masks2.shape[1]
    f = out_b.shape[1]
    t = wt.shape[1]
    o = ww.shape[1]
    bb = 200 if b % 200 == 0 else 8
    nb = b // bb
    bbk = bb * k
    feat_off = (b * k) // bb

    body = functools.partial(_fused_body, bb, k, p, t, f, nb, b)
    full = lambda i: (0, 0)
    out, tout, lfilm = pl.pallas_call(
        body,
        grid=(nb,),
        in_specs=[
            pl.BlockSpec((bbk, f), lambda i: (i, 0))
            for _ in range(p)
        ] + [
            pl.BlockSpec((bbk, f), lambda i: (i, 0)),
            pl.BlockSpec((bb, f), lambda i: (feat_off + i, 0)),
            pl.BlockSpec((bbk, p), lambda i: (i, 0)),
            pl.BlockSpec((bb, k), lambda i: (i, 0)),
            pl.BlockSpec(wt.shape, full),
            pl.BlockSpec(bt.shape, full),
            pl.BlockSpec(wg.shape, full),
            pl.BlockSpec(bg.shape, full),
            pl.BlockSpec(wb.shape, full),
            pl.BlockSpec(bbias.shape, full),
            pl.BlockSpec(wav.shape, full),
            pl.BlockSpec(bav.shape, full),
            pl.BlockSpec(wac.shape, full),
            pl.BlockSpec(bac.shape, full),
            pl.BlockSpec(wgate.shape, full),
            pl.BlockSpec(bgate.shape, full),
            pl.BlockSpec(wpa.shape, full),
            pl.BlockSpec(ww.shape, full),
            pl.BlockSpec(bw.shape, full),
        ],
        out_specs=[
            pl.BlockSpec((bb, o), lambda i: (i, 0)),
            pl.BlockSpec((bb, t), lambda i: (i, 0)),
            pl.BlockSpec((1, 1), full),
        ],
        out_shape=[
            jax.ShapeDtypeStruct((b, o), jnp.float32),
            jax.ShapeDtypeStruct((b, t), jnp.float32),
            jax.ShapeDtypeStruct((1, 1), jnp.float32),
        ],
    )(*out_p, out_b, out_b, masks2, lp, wt, bt, wg, bg, wb, bbias, wav, bav,
      wac, bac, wgate, bgate, wpa, ww, bw)
    return out, tout, lfilm


def kernel(embed, idx, paths, masks, neighs, W_w, b_w, W_t, b_t, W_g, b_g,
           W_b, b_b, W_av, b_av, W_ac, b_ac, W_gate, b_gate, W_pa, b_pa):
    n, f = embed.shape
    b, k, p = paths.shape
    t = W_t.shape[1]

    unit_a = _NW * _CHUNK * _NBUF
    r_a = ((b * k + unit_a - 1) // unit_a) * unit_a
    unit_b = _NW * _CB * _NBUF
    r_b = ((b * k + b + unit_b - 1) // unit_b) * unit_b
    zpad_a = jnp.zeros((r_a - b * k,), jnp.int32)
    idx_p = [jnp.concatenate([paths[:, :, j].reshape(-1), zpad_a])
             for j in range(p)]
    idx_b = jnp.concatenate(
        [neighs[:, :, 0].reshape(-1), idx,
         jnp.zeros((r_b - b * k - b,), jnp.int32)])
    *out_p, out_b = _sc_gather_all(embed, idx_p, idx_b, r_a, r_b)

    masks2 = masks.reshape(b * k, p)
    lp = neighs[:, :, 1].astype(jnp.float32)

    out, tout, lfilm = _fused(
        out_p, out_b, masks2, lp,
        W_t, b_t.reshape(1, t),
        W_g, b_g.reshape(1, f), W_b, b_b.reshape(1, f),
        W_av, b_av.reshape(1, -1), W_ac, b_ac.reshape(1, f),
        W_gate, b_gate.reshape(1, 1), W_pa,
        W_w, b_w.reshape(1, -1),
    )
    return out, tout, lfilm[0, 0]


# speedup vs baseline: 1.2498x; 1.2498x over previous
import functools

import jax
import jax.numpy as jnp
from jax import lax
from jax.experimental import pallas as pl
from jax.experimental.pallas import tpu as pltpu
from jax.experimental.pallas import tpu_sc as plsc

_LAMDA = 1e-05
_SLOPE = 0.01

_NC = 2
_NS = 16
_NW = _NC * _NS
_CHUNK = 128


def _leaky(x):
    return jnp.where(x >= 0, x, _SLOPE * x)


_NBUF = 5
_CB = 120


def _sc_gather_all(table, idx_p, idx_b, r_a, r_b):
    p = len(idx_p)
    d = table.shape[1]
    rw_a = r_a // _NW
    nch_a = rw_a // _CHUNK
    rw_b = r_b // _NW
    nch_b = rw_b // _CB
    assert nch_a % _NBUF == 0 and nch_b % _NBUF == 0

    mesh = plsc.VectorSubcoreMesh(
        core_axis_name="c", subcore_axis_name="s",
        num_cores=_NC, num_subcores=_NS,
    )

    @functools.partial(
        pl.kernel,
        out_type=(
            [jax.ShapeDtypeStruct((r_a, d), table.dtype)] * p
            + [jax.ShapeDtypeStruct((r_b, d), table.dtype)]
        ),
        mesh=mesh,
        scratch_types=(
            [pltpu.VMEM((p * rw_a,), jnp.int32),
             pltpu.VMEM((rw_b,), jnp.int32)]
            + [pltpu.VMEM((_CHUNK, d), table.dtype)] * _NBUF
            + [pltpu.SemaphoreType.DMA] * (2 * _NBUF)
        ),
    )
    def gather_k(table_hbm, *rest):
        idxp_hbm = rest[:p]
        idxb_hbm = rest[p]
        outp_hbm = rest[p + 1:2 * p + 1]
        outb_hbm = rest[2 * p + 1]
        sc = rest[2 * p + 2:]
        idxa_v, idxb_v = sc[0], sc[1]
        bufs = sc[2:2 + _NBUF]
        semg = sc[2 + _NBUF:2 + 2 * _NBUF]
        semw = sc[2 + 2 * _NBUF:]
        wid = lax.axis_index("s") * _NC + lax.axis_index("c")
        base_a = wid * rw_a
        base_b = wid * rw_b

        for j in range(p):
            pltpu.sync_copy(idxp_hbm[j].at[pl.ds(base_a, rw_a)],
                            idxa_v.at[pl.ds(j * rw_a, rw_a)])
        pltpu.sync_copy(idxb_hbm.at[pl.ds(base_b, rw_b)], idxb_v)

        def pipe(nch, gather_mk, write_mk):
            ng = nch // _NBUF
            for b in range(_NBUF):
                gather_mk(b, b).start()

            def body(g, carry):
                for b in range(_NBUF):
                    ch = g * _NBUF + b
                    gather_mk(ch, b).wait()
                    write_mk(ch, b).start()

                @pl.when(g < ng - 1)
                def _():
                    for b in range(_NBUF):
                        ch = g * _NBUF + b
                        write_mk(ch, b).wait()
                        gather_mk(ch + _NBUF, b).start()

                return carry

            lax.fori_loop(0, ng, body, 0)
            for b in range(_NBUF):
                write_mk((ng - 1) * _NBUF + b, b).wait()

        for j in range(p):
            def ga(ch, b, j=j):
                return pltpu.make_async_copy(
                    table_hbm.at[idxa_v.at[
                        pl.ds(j * rw_a + ch * _CHUNK, _CHUNK)]],
                    bufs[b], semg[b])

            def wa(ch, b, j=j):
                return pltpu.make_async_copy(
                    bufs[b],
                    outp_hbm[j].at[pl.ds(base_a + ch * _CHUNK, _CHUNK)],
                    semw[b])

            pipe(nch_a, ga, wa)

        def gb(ch, b):
            return pltpu.make_async_copy(
                table_hbm.at[idxb_v.at[pl.ds(ch * _CB, _CB)]],
                bufs[b].at[pl.ds(0, _CB)], semg[b])

        def wb(ch, b):
            return pltpu.make_async_copy(
                bufs[b].at[pl.ds(0, _CB)],
                outb_hbm.at[pl.ds(base_b + ch * _CB, _CB)], semw[b])

        pipe(nch_b, gb, wb)

    return gather_k(table, *idx_p, idx_b)


def _fused_body(bb, k, p, t, f, nb, b_total, *refs):
    (raw_refs, (hl_ref, feat_ref, m_ref, lp_ref,
                wt_ref, bt_ref,
                wg_ref, bg_ref, wb_ref, bbias_ref, wav_ref, bav_ref,
                wac_ref, bac_ref, wgate_ref, bgate_ref, wpa_ref,
                ww_ref, bw_ref, out_ref, tout_ref, l_ref)) = (
        refs[:p], refs[p:])
    i = pl.program_id(0)
    bbk = bb * k
    f32 = jnp.float32

    wt = wt_ref[...]
    bt = bt_ref[...]
    wtpa = jnp.dot(wt, wpa_ref[...], preferred_element_type=f32)
    segs = [r[...] for r in raw_refs]
    scores = jnp.concatenate(
        [jnp.dot(s, wtpa, preferred_element_type=f32) for s in segs], axis=1
    )
    smax = jnp.max(scores, axis=1, keepdims=True)
    e = jnp.exp(scores - smax)
    pa = e / jnp.sum(e, axis=1, keepdims=True)
    m = m_ref[...]
    w = pa * m
    z = segs[0] * w[:, 0:1]
    for j in range(1, p):
        z = z + segs[j] * w[:, j:j + 1]
    wsum = jnp.sum(w, axis=1, keepdims=True)
    t_p = (jnp.dot(z, wt, preferred_element_type=f32) + bt * wsum) / (
        jnp.sum(m, axis=1, keepdims=True) + 1e-10)

    gamma = _leaky(jnp.dot(t_p, wg_ref[...], preferred_element_type=f32)
                   + bg_ref[...])
    beta = _leaky(jnp.dot(t_p, wb_ref[...], preferred_element_type=f32)
                  + bbias_ref[...])
    hl = hl_ref[...]
    p_x = (gamma + 1.0) * hl + beta

    gsel = (lax.broadcasted_iota(jnp.int32, (bb, bbk), 1) // k
            == lax.broadcasted_iota(jnp.int32, (bb, bbk), 0)).astype(f32)
    rsel = (lax.broadcasted_iota(jnp.int32, (bbk, bb), 0) // k
            == lax.broadcasted_iota(jnp.int32, (bbk, bb), 1)).astype(f32)
    onek = (lax.broadcasted_iota(jnp.int32, (bbk, k), 0) % k
            == lax.broadcasted_iota(jnp.int32, (bbk, k), 1)).astype(f32)
    tsel = (lax.broadcasted_iota(jnp.int32, (k, bbk), 1) % k
            == lax.broadcasted_iota(jnp.int32, (k, bbk), 0)).astype(f32)

    gsum = jnp.dot(gsel, gamma * gamma, preferred_element_type=f32)
    bsum = jnp.dot(gsel, beta * beta, preferred_element_type=f32)
    partial = jnp.sum(jnp.sqrt(gsum)) + jnp.sum(jnp.sqrt(bsum))

    feat = feat_ref[...]
    bav = bav_ref[...]
    q = jnp.dot(feat, wav_ref[...], preferred_element_type=f32) + bav
    katt = jnp.dot(p_x, wav_ref[...], preferred_element_type=f32) + bav
    q_rep = jnp.dot(rsel, q, preferred_element_type=f32)
    s_col = jnp.sum(q_rep * katt, axis=1, keepdims=True)
    s_mat = jnp.dot(gsel, s_col * onek, preferred_element_type=f32)
    amax = jnp.max(s_mat, axis=1, keepdims=True)
    ae = jnp.exp(s_mat - amax)
    aw = ae / jnp.sum(ae, axis=1, keepdims=True)

    alpha = jnp.exp(-_LAMDA * lp_ref[...])
    waw = jnp.dot(aw, tsel, preferred_element_type=f32) * gsel
    wal = jnp.dot(alpha, tsel, preferred_element_type=f32) * gsel
    attn_in = jnp.dot(waw, p_x, preferred_element_type=f32)
    a_x = jnp.dot(wal, p_x, preferred_element_type=f32)

    attn_output = jnp.dot(attn_in, wac_ref[...],
                          preferred_element_type=f32) + bac_ref[...]
    g1 = wgate_ref[0:f, :]
    g2 = wgate_ref[f:2 * f, :]
    gate = jax.nn.sigmoid(
        jnp.dot(a_x, g1, preferred_element_type=f32)
        + jnp.dot(attn_output, g2, preferred_element_type=f32)
        + bgate_ref[...]
    )
    fused = gate * a_x + (1.0 - gate) * attn_output
    update = feat + fused
    out_ref[...] = _leaky(jnp.dot(update, ww_ref[...],
                                  preferred_element_type=f32) + bw_ref[...])
    tout_ref[...] = jnp.dot(feat, wt, preferred_element_type=f32) + bt

    prev = jnp.where(i == 0, jnp.zeros((1, 1), f32), l_ref[...])
    tot = prev + partial
    l_ref[...] = jnp.where(i == nb - 1, tot / b_total, tot)


def _fused(out_p, out_b, masks2, lp, wt, bt, wg, bg, wb, bbias,
           wav, bav, wac, bac, wgate, bgate, wpa, ww, bw):
    b, k = lp.shape
    p = masks2.shape[1]
    f = out_b.shape[1]
    t = wt.shape[1]
    o = ww.shape[1]
    bb = 200 if b % 200 == 0 else 8
    nb = b // bb
    bbk = bb * k
    feat_off = (b * k) // bb

    body = functools.partial(_fused_body, bb, k, p, t, f, nb, b)
    full = lambda i: (0, 0)
    out, tout, lfilm = pl.pallas_call(
        body,
        grid=(nb,),
        in_specs=[
            pl.BlockSpec((bbk, f), lambda i: (i, 0))
            for _ in range(p)
        ] + [
            pl.BlockSpec((bbk, f), lambda i: (i, 0)),
            pl.BlockSpec((bb, f), lambda i: (feat_off + i, 0)),
            pl.BlockSpec((bbk, p), lambda i: (i, 0)),
            pl.BlockSpec((bb, k), lambda i: (i, 0)),
            pl.BlockSpec(wt.shape, full),
            pl.BlockSpec(bt.shape, full),
            pl.BlockSpec(wg.shape, full),
            pl.BlockSpec(bg.shape, full),
            pl.BlockSpec(wb.shape, full),
            pl.BlockSpec(bbias.shape, full),
            pl.BlockSpec(wav.shape, full),
            pl.BlockSpec(bav.shape, full),
            pl.BlockSpec(wac.shape, full),
            pl.BlockSpec(bac.shape, full),
            pl.BlockSpec(wgate.shape, full),
            pl.BlockSpec(bgate.shape, full),
            pl.BlockSpec(wpa.shape, full),
            pl.BlockSpec(ww.shape, full),
            pl.BlockSpec(bw.shape, full),
        ],
        out_specs=[
            pl.BlockSpec((bb, o), lambda i: (i, 0)),
            pl.BlockSpec((bb, t), lambda i: (i, 0)),
            pl.BlockSpec((1, 1), full),
        ],
        out_shape=[
            jax.ShapeDtypeStruct((b, o), jnp.float32),
            jax.ShapeDtypeStruct((b, t), jnp.float32),
            jax.ShapeDtypeStruct((1, 1), jnp.float32),
        ],
    )(*out_p, out_b, out_b, masks2, lp, wt, bt, wg, bg, wb, bbias, wav, bav,
      wac, bac, wgate, bgate, wpa, ww, bw)
    return out, tout, lfilm


def kernel(embed, idx, paths, masks, neighs, W_w, b_w, W_t, b_t, W_g, b_g,
           W_b, b_b, W_av, b_av, W_ac, b_ac, W_gate, b_gate, W_pa, b_pa):
    n, f = embed.shape
    b, k, p = paths.shape
    t = W_t.shape[1]

    unit_a = _NW * _CHUNK * _NBUF
    r_a = ((b * k + unit_a - 1) // unit_a) * unit_a
    unit_b = _NW * _CB * _NBUF
    r_b = ((b * k + b + unit_b - 1) // unit_b) * unit_b
    zpad_a = jnp.zeros((r_a - b * k,), jnp.int32)
    idx_p = [jnp.concatenate([paths[:, :, j].reshape(-1), zpad_a])
             for j in range(p)]
    idx_b = jnp.concatenate(
        [neighs[:, :, 0].reshape(-1), idx,
         jnp.zeros((r_b - b * k - b,), jnp.int32)])
    *out_p, out_b = _sc_gather_all(embed, idx_p, idx_b, r_a, r_b)

    masks2 = masks.reshape(b * k, p)
    lp = neighs[:, :, 1].astype(jnp.float32)

    out, tout, lfilm = _fused(
        out_p, out_b, masks2, lp,
        W_t, b_t.reshape(1, t),
        W_g, b_g.reshape(1, f), W_b, b_b.reshape(1, f),
        W_av, b_av.reshape(1, -1), W_ac, b_ac.reshape(1, f),
        W_gate, b_gate.reshape(1, 1), W_pa,
        W_w, b_w.reshape(1, -1),
    )
    return out, tout, lfilm[0, 0]
